# baseline (speedup 1.0000x reference)
import functools

import jax
import jax.numpy as jnp
from jax import lax
from jax.experimental import pallas as pl
from jax.experimental.pallas import tpu as pltpu
from jax.experimental.pallas import tpu_sc as plsc

_N, _E, _DIN, _DH, _DOUT = 10000, 320000, 128, 64, 2
_R = 2000
_NBLK = _N // _R

_TPC = 16
_CHW = 128
_NCH = 160
_EPAD = _TPC * _NCH * _CHW
_NPAD = 10240
_RPT = _NPAD // _TPC
_GRT = _N // _TPC
_DHH = _DH // 2
_NBUF = 8
_DEGC = _NCH // 2

_f32 = jnp.float32


def _mm1_body(x_ref, w_ref, h_ref):
    h_ref[...] = jnp.dot(x_ref[...], w_ref[...],
                         preferred_element_type=jnp.float32)


def _stage1_body(deg_ref, h_ref, dinv_ref, g_ref):
    deg = deg_ref[0] + deg_ref[1] + 1.0
    dinv = jax.lax.rsqrt(deg)
    dinv_ref[...] = dinv
    g_ref[...] = h_ref[...] * dinv


def _mid_body(s_ref, g_ref, dinv_ref, b_ref, w_ref, gout_ref):
    dinv = dinv_ref[...]
    s = s_ref[...] + g_ref[...]
    xn = jnp.maximum(dinv * s + b_ref[...], 0.0)
    h = jnp.dot(xn, w_ref[...], preferred_element_type=jnp.float32)
    gout_ref[...] = h * dinv


def _final_body(s_ref, g_ref, dinv_ref, b_ref, wfc_ref, bfc_ref, out_ref):
    dinv = dinv_ref[...]
    s = s_ref[...] + g_ref[...]
    xn = jnp.maximum(dinv * s + b_ref[...], 0.0)
    logits = jnp.dot(xn, wfc_ref[...], preferred_element_type=jnp.float32)
    logits = logits + bfc_ref[...]
    m = jnp.max(logits, axis=1, keepdims=True)
    lse = jnp.log(jnp.sum(jnp.exp(logits - m), axis=1, keepdims=True)) + m
    out_ref[...] = logits - lse


def _rows(i):
    return (i, 0)


def _rows3(i):
    return (0, i, 0)


def _rep(i):
    return (0, 0)


def _mm1(x, w1):
    return pl.pallas_call(
        _mm1_body,
        grid=(_NBLK,),
        in_specs=[
            pl.BlockSpec((_R, _DIN), _rows),
            pl.BlockSpec((_DIN, _DH), _rep),
        ],
        out_specs=pl.BlockSpec((_R, _DH), _rows),
        out_shape=jax.ShapeDtypeStruct((_N, _DH), _f32),
    )(x, w1)


def _stage1(degs, h1):
    return pl.pallas_call(
        _stage1_body,
        grid=(_NBLK,),
        in_specs=[
            pl.BlockSpec((2, _R, 1), _rows3),
            pl.BlockSpec((_R, _DH), _rows),
        ],
        out_specs=[
            pl.BlockSpec((_R, 1), _rows),
            pl.BlockSpec((_R, _DH), _rows),
        ],
        out_shape=[
            jax.ShapeDtypeStruct((_N, 1), _f32),
            jax.ShapeDtypeStruct((_N, _DH), _f32),
        ],
    )(degs, h1)


def _mid(s, g, dinv, b, w):
    return pl.pallas_call(
        _mid_body,
        grid=(_NBLK,),
        in_specs=[
            pl.BlockSpec((_R, _DH), _rows),
            pl.BlockSpec((_R, _DH), _rows),
            pl.BlockSpec((_R, 1), _rows),
            pl.BlockSpec((1, _DH), _rep),
            pl.BlockSpec((_DH, _DH), _rep),
        ],
        out_specs=pl.BlockSpec((_R, _DH), _rows),
        out_shape=jax.ShapeDtypeStruct((_N, _DH), _f32),
    )(s, g, dinv, b, w)


def _final(s, g, dinv, b, wfc, bfc):
    return pl.pallas_call(
        _final_body,
        grid=(_NBLK,),
        in_specs=[
            pl.BlockSpec((_R, _DH), _rows),
            pl.BlockSpec((_R, _DH), _rows),
            pl.BlockSpec((_R, 1), _rows),
            pl.BlockSpec((1, _DH), _rep),
            pl.BlockSpec((_DH, _DOUT), _rep),
            pl.BlockSpec((1, _DOUT), _rep),
        ],
        out_specs=pl.BlockSpec((_R, _DOUT), _rows),
        out_shape=jax.ShapeDtypeStruct((_N, _DOUT), _f32),
    )(s, g, dinv, b, wfc, bfc)


_MESH = plsc.VectorSubcoreMesh(core_axis_name="c", subcore_axis_name="s")
_SC_PARAMS = pltpu.CompilerParams(use_tc_tiling_on_sc=False)


def _drain(sem, src, dst):
    pltpu.make_async_copy(src, dst, sem).wait()


def _sc_deg(dst_r, z1d):
    @functools.partial(
        pl.kernel,
        out_type=jax.ShapeDtypeStruct((2, _NPAD), _f32),
        mesh=_MESH,
        compiler_params=_SC_PARAMS,
        scratch_types=[
            pltpu.VMEM((_DEGC, _CHW), jnp.int32),
            pltpu.VMEM((_CHW,), _f32),
            pltpu.VMEM_SHARED((_NPAD,), _f32),
            pltpu.SemaphoreType.DMA,
        ],
    )
    def k(dst_hbm, z1d_hbm, out_hbm, idx_d, ones_v, acc, ssem):
        cid = lax.axis_index("c")
        sid = lax.axis_index("s")
        pltpu.sync_copy(dst_hbm.at[sid, pl.ds(cid * _DEGC, _DEGC)], idx_d)
        for i in range(_CHW // 16):
            ones_v[pl.ds(i * 16, 16)] = jnp.ones((16,), _f32)
        r0 = sid * _RPT
        pltpu.sync_copy(z1d_hbm.at[pl.ds(r0, _RPT)], acc.at[pl.ds(r0, _RPT)])
        plsc.subcore_barrier()

        @pl.loop(0, _DEGC)
        def _(j):
            pltpu.async_copy(ones_v, acc.at[idx_d.at[j]], ssem, add=True)

        @pl.loop(0, _DEGC)
        def _(j):
            _drain(ssem, z1d_hbm.at[pl.ds(0, _CHW)], ones_v)

        plsc.subcore_barrier()
        pltpu.sync_copy(acc.at[pl.ds(r0, _RPT)],
                        out_hbm.at[cid, pl.ds(r0, _RPT)])

    return k(dst_r, z1d)


def _sc_scatter(src_r, dst_r, g, z2d):
    @functools.partial(
        pl.kernel,
        out_type=pltpu.HBM((_NPAD, _DH), _f32),
        mesh=_MESH,
        compiler_params=_SC_PARAMS,
        scratch_types=[
            pltpu.VMEM((_NCH, _CHW), jnp.int32),
            pltpu.VMEM((_NCH, _CHW), jnp.int32),
            [pltpu.VMEM((_CHW, _DHH), _f32) for _ in range(_NBUF)],
            pltpu.VMEM_SHARED((_N, _DHH), _f32),
            pltpu.VMEM_SHARED((_NPAD, _DHH), _f32),
            [pltpu.SemaphoreType.DMA for _ in range(_NBUF)],
            [pltpu.SemaphoreType.DMA for _ in range(_NBUF)],
        ],
    )
    def k(src_hbm, dst_hbm, g_hbm, z2d_hbm, out_hbm,
          idx_s, idx_d, bufs, gsp, acc, gsems, ssems):
        cid = lax.axis_index("c")
        sid = lax.axis_index("s")
        pltpu.sync_copy(src_hbm.at[sid], idx_s)
        pltpu.sync_copy(dst_hbm.at[sid], idx_d)
        gr0 = sid * _GRT
        pltpu.sync_copy(g_hbm.at[pl.ds(gr0, _GRT), pl.ds(cid * _DHH, _DHH)],
                        gsp.at[pl.ds(gr0, _GRT)])
        r0 = sid * _RPT
        pltpu.sync_copy(z2d_hbm, acc.at[pl.ds(r0, _RPT)])
        plsc.subcore_barrier()

        for kk in range(_NBUF):
            pltpu.async_copy(gsp.at[idx_s.at[kk]], bufs[kk], gsems[kk])

        @pl.loop(0, _NCH, step=_NBUF)
        def _(j):
            for kk in range(_NBUF):
                _drain(gsems[kk], g_hbm.at[pl.ds(0, _CHW), pl.ds(0, _DHH)],
                       bufs[kk])
                pltpu.async_copy(bufs[kk], acc.at[idx_d.at[j + kk]],
                                 ssems[kk], add=True)
            for kk in range(_NBUF):
                jn = j + _NBUF + kk

                @pl.when(jn < _NCH)
                def _():
                    _drain(ssems[kk], g_hbm.at[pl.ds(0, _CHW), pl.ds(0, _DHH)],
                           bufs[kk])
                    pltpu.async_copy(gsp.at[idx_s.at[jn]], bufs[kk],
                                     gsems[kk])

        for kk in range(_NBUF):
            _drain(ssems[kk], g_hbm.at[pl.ds(0, _CHW), pl.ds(0, _DHH)],
                   bufs[kk])
        plsc.subcore_barrier()
        pltpu.sync_copy(acc.at[pl.ds(r0, _RPT)],
                        out_hbm.at[pl.ds(r0, _RPT), pl.ds(cid * _DHH, _DHH)])

    return k(src_r, dst_r, g, z2d)


def kernel(x, edge_index, W1, b1, W2, b2, W3, b3, Wfc, bfc):
    src, dst = edge_index[0], edge_index[1]
    npad = _EPAD - _E
    pidx = jnp.arange(npad, dtype=jnp.int32)
    src_p = jnp.concatenate([src, (pidx * 37) % _N]).reshape(_TPC, _NCH, _CHW)
    dst_p = jnp.concatenate([dst, _N + pidx % (_NPAD - _N)]).reshape(
        _TPC, _NCH, _CHW)
    z1d = jnp.zeros((_NPAD,), _f32)
    z2d = jnp.zeros((_RPT, _DHH), _f32)

    h1 = _mm1(x, W1)
    degs = _sc_deg(dst_p, z1d)[:, :_N].reshape(2, _N, 1)
    dinv, g1 = _stage1(degs, h1)
    s1 = _sc_scatter(src_p, dst_p, g1, z2d)[:_N]
    g2 = _mid(s1, g1, dinv, b1[None, :], W2)
    s2 = _sc_scatter(src_p, dst_p, g2, z2d)[:_N]
    g3 = _mid(s2, g2, dinv, b2[None, :], W3)
    s3 = _sc_scatter(src_p, dst_p, g3, z2d)[:_N]
    return _final(s3, g3, dinv, b3[None, :], Wfc, bfc[None, :])


# speedup vs baseline: 1.3459x; 1.3459x over previous
import functools

import jax
import jax.numpy as jnp
from jax import lax
from jax.experimental import pallas as pl
from jax.experimental.pallas import tpu as pltpu
from jax.experimental.pallas import tpu_sc as plsc

_N, _E, _DIN, _DH, _DOUT = 10000, 320000, 128, 64, 2
_NH = _N // 2
_R = 1000
_NBLK = _NH // _R

_NW = 32
_TPC = 16
_CHW = 128
_NCH = 80
_EPAD = _NW * _NCH * _CHW
_NPAD = 10240
_RPT = _NPAD // _TPC
_NBUF = 8

_f32 = jnp.float32


def _mm1_body(xlo_ref, xhi_ref, w_ref, h_ref):
    w = w_ref[...]
    hlo = jnp.dot(xlo_ref[...], w, preferred_element_type=jnp.float32)
    hhi = jnp.dot(xhi_ref[...], w, preferred_element_type=jnp.float32)
    h_ref[...] = jnp.concatenate([hlo, hhi], axis=1)


def _stage1_body(deg_ref, h_ref, dinv_ref, g_ref):
    deg = deg_ref[0] + deg_ref[1] + 1.0
    dinv = jax.lax.rsqrt(deg)
    dlo = jnp.broadcast_to(dinv[:, 0:1], (_R, _DH))
    dhi = jnp.broadcast_to(dinv[:, 1:2], (_R, _DH))
    dp = jnp.concatenate([dlo, dhi], axis=1)
    dinv_ref[...] = dp
    g_ref[...] = h_ref[...] * dp


def _mid_body(s_ref, g_ref, dinv_ref, b_ref, w_ref, gout_ref):
    dp = dinv_ref[...]
    s = s_ref[0] + s_ref[1] + g_ref[...]
    xn = jnp.maximum(dp * s + b_ref[...], 0.0)
    h = jnp.dot(xn, w_ref[...], preferred_element_type=jnp.float32)
    gout_ref[...] = h * dp


def _final_body(s_ref, g_ref, dinv_ref, b_ref, wfc_ref, bfc_ref, out_ref):
    dp = dinv_ref[...]
    s = s_ref[0] + s_ref[1] + g_ref[...]
    xn = jnp.maximum(dp * s + b_ref[...], 0.0)
    logits = jnp.dot(xn, wfc_ref[...], preferred_element_type=jnp.float32)
    logits = logits + bfc_ref[...]
    la, lb = logits[:, 0:2], logits[:, 2:4]
    ma = jnp.max(la, axis=1, keepdims=True)
    mb = jnp.max(lb, axis=1, keepdims=True)
    za = jnp.log(jnp.sum(jnp.exp(la - ma), axis=1, keepdims=True)) + ma
    zb = jnp.log(jnp.sum(jnp.exp(lb - mb), axis=1, keepdims=True)) + mb
    out_ref[...] = jnp.concatenate([la - za, lb - zb], axis=1)


def _rows(i):
    return (i, 0)


def _rows_hi(i):
    return (i + _NBLK, 0)


def _rows3(i):
    return (0, i, 0)


def _rep(i):
    return (0, 0)


def _rep3(i):
    return (0, 0, 0)


def _mm1(x, w1):
    return pl.pallas_call(
        _mm1_body,
        grid=(_NBLK,),
        in_specs=[
            pl.BlockSpec((_R, _DIN), _rows),
            pl.BlockSpec((_R, _DIN), _rows_hi),
            pl.BlockSpec((_DIN, _DH), _rep),
        ],
        out_specs=pl.BlockSpec((_R, 2 * _DH), _rows),
        out_shape=jax.ShapeDtypeStruct((_NH, 2 * _DH), _f32),
    )(x, x, w1)


def _stage1(degs, h1):
    return pl.pallas_call(
        _stage1_body,
        grid=(_NBLK,),
        in_specs=[
            pl.BlockSpec((2, _R, 2), _rows3),
            pl.BlockSpec((_R, 2 * _DH), _rows),
        ],
        out_specs=[
            pl.BlockSpec((_R, 2 * _DH), _rows),
            pl.BlockSpec((_R, 2 * _DH), _rows),
        ],
        out_shape=[
            jax.ShapeDtypeStruct((_NH, 2 * _DH), _f32),
            jax.ShapeDtypeStruct((_NH, 2 * _DH), _f32),
        ],
    )(degs, h1)


def _mid(s, g, dinv, b, w):
    return pl.pallas_call(
        _mid_body,
        grid=(_NBLK,),
        in_specs=[
            pl.BlockSpec((2, _R, 2 * _DH), _rows3),
            pl.BlockSpec((_R, 2 * _DH), _rows),
            pl.BlockSpec((_R, 2 * _DH), _rows),
            pl.BlockSpec((1, 2 * _DH), _rep),
            pl.BlockSpec((2 * _DH, 2 * _DH), _rep),
        ],
        out_specs=pl.BlockSpec((_R, 2 * _DH), _rows),
        out_shape=jax.ShapeDtypeStruct((_NH, 2 * _DH), _f32),
    )(s, g, dinv, b, w)


def _final(s, g, dinv, b, wfc, bfc):
    return pl.pallas_call(
        _final_body,
        grid=(_NBLK,),
        in_specs=[
            pl.BlockSpec((2, _R, 2 * _DH), _rows3),
            pl.BlockSpec((_R, 2 * _DH), _rows),
            pl.BlockSpec((_R, 2 * _DH), _rows),
            pl.BlockSpec((1, 2 * _DH), _rep),
            pl.BlockSpec((2 * _DH, 2 * _DOUT), _rep),
            pl.BlockSpec((1, 2 * _DOUT), _rep),
        ],
        out_specs=pl.BlockSpec((_R, 2 * _DOUT), _rows),
        out_shape=jax.ShapeDtypeStruct((_NH, 2 * _DOUT), _f32),
    )(s, g, dinv, b, wfc, bfc)


_MESH = plsc.VectorSubcoreMesh(core_axis_name="c", subcore_axis_name="s")
_SC_PARAMS = pltpu.CompilerParams(use_tc_tiling_on_sc=False)


def _drain(sem, src, dst):
    pltpu.make_async_copy(src, dst, sem).wait()


def _sc_deg(dst_r, z1d):
    @functools.partial(
        pl.kernel,
        out_type=jax.ShapeDtypeStruct((2, _NPAD), _f32),
        mesh=_MESH,
        compiler_params=_SC_PARAMS,
        scratch_types=[
            pltpu.VMEM((_NCH, _CHW), jnp.int32),
            pltpu.VMEM((_CHW,), _f32),
            pltpu.VMEM_SHARED((_NPAD,), _f32),
            pltpu.SemaphoreType.DMA,
        ],
    )
    def k(dst_hbm, z1d_hbm, out_hbm, idx_d, ones_v, acc, ssem):
        cid = lax.axis_index("c")
        sid = lax.axis_index("s")
        wid = cid * _TPC + sid
        pltpu.sync_copy(dst_hbm.at[wid], idx_d)
        for i in range(_CHW // 16):
            ones_v[pl.ds(i * 16, 16)] = jnp.ones((16,), _f32)
        r0 = sid * _RPT
        pltpu.sync_copy(z1d_hbm.at[pl.ds(r0, _RPT)], acc.at[pl.ds(r0, _RPT)])
        plsc.subcore_barrier()

        @pl.loop(0, _NCH)
        def _(j):
            pltpu.async_copy(ones_v, acc.at[idx_d.at[j]], ssem, add=True)

        @pl.loop(0, _NCH)
        def _(j):
            _drain(ssem, z1d_hbm.at[pl.ds(0, _CHW)], ones_v)

        plsc.subcore_barrier()
        pltpu.sync_copy(acc.at[pl.ds(r0, _RPT)],
                        out_hbm.at[cid, pl.ds(r0, _RPT)])

    return k(dst_r, z1d)


def _sc_scatter(src_r, dst_r, g, z2d):
    @functools.partial(
        pl.kernel,
        out_type=jax.ShapeDtypeStruct((2, _NPAD, _DH), _f32),
        mesh=_MESH,
        compiler_params=_SC_PARAMS,
        scratch_types=[
            pltpu.VMEM((_NCH, _CHW), jnp.int32),
            pltpu.VMEM((_NCH, _CHW), jnp.int32),
            [pltpu.VMEM((_CHW, _DH), _f32) for _ in range(_NBUF)],
            pltpu.VMEM_SHARED((_NPAD, _DH), _f32),
            [pltpu.SemaphoreType.DMA for _ in range(_NBUF)],
            [pltpu.SemaphoreType.DMA for _ in range(_NBUF)],
        ],
    )
    def k(src_hbm, dst_hbm, g_hbm, z2d_hbm, out_hbm,
          idx_s, idx_d, bufs, acc, gsems, ssems):
        cid = lax.axis_index("c")
        sid = lax.axis_index("s")
        wid = cid * _TPC + sid
        pltpu.sync_copy(src_hbm.at[wid], idx_s)
        pltpu.sync_copy(dst_hbm.at[wid], idx_d)
        r0 = sid * _RPT
        pltpu.sync_copy(z2d_hbm, acc.at[pl.ds(r0, _RPT)])
        plsc.subcore_barrier()

        for kk in range(_NBUF):
            pltpu.async_copy(g_hbm.at[idx_s.at[kk]], bufs[kk], gsems[kk])

        @pl.loop(0, _NCH, step=_NBUF)
        def _(j):
            for kk in range(_NBUF):
                _drain(gsems[kk], g_hbm.at[pl.ds(0, _CHW)], bufs[kk])
                pltpu.async_copy(bufs[kk], acc.at[idx_d.at[j + kk]],
                                 ssems[kk], add=True)
            for kk in range(_NBUF):
                jn = j + _NBUF + kk

                @pl.when(jn < _NCH)
                def _():
                    _drain(ssems[kk], g_hbm.at[pl.ds(0, _CHW)], bufs[kk])
                    pltpu.async_copy(g_hbm.at[idx_s.at[jn]], bufs[kk],
                                     gsems[kk])

        for kk in range(_NBUF):
            _drain(ssems[kk], g_hbm.at[pl.ds(0, _CHW)], bufs[kk])
        plsc.subcore_barrier()
        pltpu.sync_copy(acc.at[pl.ds(r0, _RPT)],
                        out_hbm.at[cid, pl.ds(r0, _RPT)])

    return k(src_r, dst_r, g, z2d)


def _blockdiag(w):
    z = jnp.zeros(w.shape, w.dtype)
    return jnp.concatenate(
        [jnp.concatenate([w, z], axis=1), jnp.concatenate([z, w], axis=1)],
        axis=0)


def kernel(x, edge_index, W1, b1, W2, b2, W3, b3, Wfc, bfc):
    src, dst = edge_index[0], edge_index[1]
    src = 2 * (src % _NH) + src // _NH
    dst = 2 * (dst % _NH) + dst // _NH
    npad = _EPAD - _E
    pidx = jnp.arange(npad, dtype=jnp.int32)
    src_p = jnp.concatenate([src, (pidx * 37) % _N]).reshape(_NW, _NCH, _CHW)
    dst_p = jnp.concatenate([dst, _N + pidx % (_NPAD - _N)]).reshape(
        _NW, _NCH, _CHW)
    z1d = jnp.zeros((_NPAD,), _f32)
    z2d = jnp.zeros((_RPT, _DH), _f32)
    w2d = _blockdiag(W2)
    w3d = _blockdiag(W3)
    wfcd = _blockdiag(Wfc)
    b1p = jnp.concatenate([b1, b1])[None, :]
    b2p = jnp.concatenate([b2, b2])[None, :]
    b3p = jnp.concatenate([b3, b3])[None, :]
    bfcp = jnp.concatenate([bfc, bfc])[None, :]

    h1 = _mm1(x, W1)
    degs = _sc_deg(dst_p, z1d).reshape(2, _NPAD // 2, 2)
    dinv, g1 = _stage1(degs, h1)

    def s_packed(g):
        s = _sc_scatter(src_p, dst_p, g.reshape(_N, _DH), z2d)
        return s.reshape(2, _NPAD // 2, 2 * _DH)

    g2 = _mid(s_packed(g1), g1, dinv, b1p, w2d)
    g3 = _mid(s_packed(g2), g2, dinv, b2p, w3d)
    outp = _final(s_packed(g3), g3, dinv, b3p, wfcd, bfcp)
    o = outp.reshape(_NH, 2, _DOUT)
    return jnp.concatenate([o[:, 0], o[:, 1]], axis=0)


# speedup vs baseline: 1.4801x; 1.0997x over previous
import functools

import jax
import jax.numpy as jnp
from jax import lax
from jax.experimental import pallas as pl
from jax.experimental.pallas import tpu as pltpu
from jax.experimental.pallas import tpu_sc as plsc

_N, _E, _DIN, _DH, _DOUT = 10000, 320000, 128, 64, 2
_NH = _N // 2
_R = 1000
_NBLK = _NH // _R

_NW = 32
_TPC = 16
_CHW = 128
_NCH = 80
_EPAD = _NW * _NCH * _CHW
_NPAD = 10240
_RPT = _NPAD // _TPC
_NBUF = 8

_f32 = jnp.float32


def _mm1_body(xlo_ref, xhi_ref, w_ref, h_ref):
    w = w_ref[...]
    hlo = jnp.dot(xlo_ref[...], w, preferred_element_type=jnp.float32)
    hhi = jnp.dot(xhi_ref[...], w, preferred_element_type=jnp.float32)
    h_ref[...] = jnp.concatenate([hlo, hhi], axis=1)


def _stage1_body(deg_ref, h_ref, dinv_ref, g_ref):
    deg = deg_ref[0] + deg_ref[1] + 1.0
    dinv = jax.lax.rsqrt(deg)
    dlo = jnp.broadcast_to(dinv[:, 0:1], (_R, _DH))
    dhi = jnp.broadcast_to(dinv[:, 1:2], (_R, _DH))
    dp = jnp.concatenate([dlo, dhi], axis=1)
    dinv_ref[...] = dp
    g_ref[...] = h_ref[...] * dp


def _mid_body(s_ref, g_ref, dinv_ref, b_ref, w_ref, gout_ref):
    dp = dinv_ref[...]
    s = s_ref[0] + s_ref[1] + g_ref[...]
    xn = jnp.maximum(dp * s + b_ref[...], 0.0)
    h = jnp.dot(xn, w_ref[...], preferred_element_type=jnp.float32)
    gout_ref[...] = h * dp


def _final_body(s_ref, g_ref, dinv_ref, b_ref, wfc_ref, bfc_ref, out_ref):
    dp = dinv_ref[...]
    s = s_ref[0] + s_ref[1] + g_ref[...]
    xn = jnp.maximum(dp * s + b_ref[...], 0.0)
    logits = jnp.dot(xn, wfc_ref[...], preferred_element_type=jnp.float32)
    logits = logits + bfc_ref[...]
    la, lb = logits[:, 0:2], logits[:, 2:4]
    ma = jnp.max(la, axis=1, keepdims=True)
    mb = jnp.max(lb, axis=1, keepdims=True)
    za = jnp.log(jnp.sum(jnp.exp(la - ma), axis=1, keepdims=True)) + ma
    zb = jnp.log(jnp.sum(jnp.exp(lb - mb), axis=1, keepdims=True)) + mb
    out_ref[...] = jnp.concatenate([la - za, lb - zb], axis=1)


def _rows(i):
    return (i, 0)


def _rows_hi(i):
    return (i + _NBLK, 0)


def _rows3(i):
    return (0, i, 0)


def _rep(i):
    return (0, 0)


def _rep3(i):
    return (0, 0, 0)


def _mm1(x, w1):
    return pl.pallas_call(
        _mm1_body,
        grid=(_NBLK,),
        in_specs=[
            pl.BlockSpec((_R, _DIN), _rows),
            pl.BlockSpec((_R, _DIN), _rows_hi),
            pl.BlockSpec((_DIN, _DH), _rep),
        ],
        out_specs=pl.BlockSpec((_R, 2 * _DH), _rows),
        out_shape=jax.ShapeDtypeStruct((_NH, 2 * _DH), _f32),
    )(x, x, w1)


def _stage1(degs, h1):
    return pl.pallas_call(
        _stage1_body,
        grid=(_NBLK,),
        in_specs=[
            pl.BlockSpec((2, _R, 2), _rows3),
            pl.BlockSpec((_R, 2 * _DH), _rows),
        ],
        out_specs=[
            pl.BlockSpec((_R, 2 * _DH), _rows),
            pl.BlockSpec((_R, 2 * _DH), _rows),
        ],
        out_shape=[
            jax.ShapeDtypeStruct((_NH, 2 * _DH), _f32),
            jax.ShapeDtypeStruct((_NH, 2 * _DH), _f32),
        ],
    )(degs, h1)


def _mid(s, g, dinv, b, w):
    return pl.pallas_call(
        _mid_body,
        grid=(_NBLK,),
        in_specs=[
            pl.BlockSpec((2, _R, 2 * _DH), _rows3),
            pl.BlockSpec((_R, 2 * _DH), _rows),
            pl.BlockSpec((_R, 2 * _DH), _rows),
            pl.BlockSpec((1, 2 * _DH), _rep),
            pl.BlockSpec((2 * _DH, 2 * _DH), _rep),
        ],
        out_specs=pl.BlockSpec((_R, 2 * _DH), _rows),
        out_shape=jax.ShapeDtypeStruct((_NH, 2 * _DH), _f32),
    )(s, g, dinv, b, w)


def _final(s, g, dinv, b, wfc, bfc):
    return pl.pallas_call(
        _final_body,
        grid=(_NBLK,),
        in_specs=[
            pl.BlockSpec((2, _R, 2 * _DH), _rows3),
            pl.BlockSpec((_R, 2 * _DH), _rows),
            pl.BlockSpec((_R, 2 * _DH), _rows),
            pl.BlockSpec((1, 2 * _DH), _rep),
            pl.BlockSpec((2 * _DH, 2 * _DOUT), _rep),
            pl.BlockSpec((1, 2 * _DOUT), _rep),
        ],
        out_specs=pl.BlockSpec((_R, 2 * _DOUT), _rows),
        out_shape=jax.ShapeDtypeStruct((_NH, 2 * _DOUT), _f32),
    )(s, g, dinv, b, wfc, bfc)


_MESH = plsc.VectorSubcoreMesh(core_axis_name="c", subcore_axis_name="s")
_SC_PARAMS = pltpu.CompilerParams(use_tc_tiling_on_sc=False)


def _drain(sem, src, dst):
    pltpu.make_async_copy(src, dst, sem).wait()


def _sc_deg(dst_r, z1d):
    @functools.partial(
        pl.kernel,
        out_type=jax.ShapeDtypeStruct((2, _NPAD), _f32),
        mesh=_MESH,
        compiler_params=_SC_PARAMS,
        scratch_types=[
            pltpu.VMEM((_NCH, _CHW), jnp.int32),
            pltpu.VMEM((_CHW,), _f32),
            pltpu.VMEM_SHARED((_NPAD,), _f32),
            pltpu.SemaphoreType.DMA,
        ],
    )
    def k(dst_hbm, z1d_hbm, out_hbm, idx_d, ones_v, acc, ssem):
        cid = lax.axis_index("c")
        sid = lax.axis_index("s")
        wid = cid * _TPC + sid
        pltpu.sync_copy(dst_hbm.at[wid], idx_d)
        for i in range(_CHW // 16):
            ones_v[pl.ds(i * 16, 16)] = jnp.ones((16,), _f32)
        r0 = sid * _RPT
        pltpu.sync_copy(z1d_hbm.at[pl.ds(r0, _RPT)], acc.at[pl.ds(r0, _RPT)])
        plsc.subcore_barrier()

        @pl.loop(0, _NCH)
        def _(j):
            pltpu.async_copy(ones_v, acc.at[idx_d.at[j]], ssem, add=True)

        @pl.loop(0, _NCH)
        def _(j):
            _drain(ssem, z1d_hbm.at[pl.ds(0, _CHW)], ones_v)

        plsc.subcore_barrier()
        pltpu.sync_copy(acc.at[pl.ds(r0, _RPT)],
                        out_hbm.at[cid, pl.ds(r0, _RPT)])

    return k(dst_r, z1d)


def _sc_scatter(src_r, dst_r, g, z2d):
    @functools.partial(
        pl.kernel,
        out_type=jax.ShapeDtypeStruct((2, _NPAD, _DH), _f32),
        mesh=_MESH,
        compiler_params=_SC_PARAMS,
        scratch_types=[
            pltpu.VMEM((_NCH, _CHW), jnp.int32),
            pltpu.VMEM((_NCH, _CHW), jnp.int32),
            [pltpu.VMEM((_CHW, _DH), _f32) for _ in range(_NBUF)],
            pltpu.VMEM_SHARED((_NPAD, _DH), _f32),
            [pltpu.SemaphoreType.DMA for _ in range(_NBUF)],
            [pltpu.SemaphoreType.DMA for _ in range(_NBUF)],
        ],
    )
    def k(src_hbm, dst_hbm, g_hbm, z2d_hbm, out_hbm,
          idx_s, idx_d, bufs, acc, gsems, ssems):
        cid = lax.axis_index("c")
        sid = lax.axis_index("s")
        wid = cid * _TPC + sid
        pltpu.sync_copy(src_hbm.at[wid], idx_s)
        pltpu.sync_copy(dst_hbm.at[wid], idx_d)
        r0 = sid * _RPT
        pltpu.sync_copy(z2d_hbm, acc.at[pl.ds(r0, _RPT)])
        plsc.subcore_barrier()

        for kk in range(_NBUF):
            pltpu.async_copy(g_hbm.at[idx_s.at[kk]], bufs[kk], gsems[kk])

        @pl.loop(0, _NCH, step=_NBUF)
        def _(j):
            for kk in range(_NBUF):
                _drain(gsems[kk], g_hbm.at[pl.ds(0, _CHW)], bufs[kk])
                pltpu.async_copy(bufs[kk], acc.at[idx_d.at[j + kk]],
                                 ssems[kk], add=True)
            for kk in range(_NBUF):
                jn = j + _NBUF + kk

                @pl.when(jn < _NCH)
                def _():
                    _drain(ssems[kk], g_hbm.at[pl.ds(0, _CHW)], bufs[kk])
                    pltpu.async_copy(g_hbm.at[idx_s.at[jn]], bufs[kk],
                                     gsems[kk])

        for kk in range(_NBUF):
            _drain(ssems[kk], g_hbm.at[pl.ds(0, _CHW)], bufs[kk])
        plsc.subcore_barrier()
        pltpu.sync_copy(acc.at[pl.ds(r0, _RPT)],
                        out_hbm.at[cid, pl.ds(r0, _RPT)])

    return k(src_r, dst_r, g, z2d)


def _blockdiag(w):
    z = jnp.zeros(w.shape, w.dtype)
    return jnp.concatenate(
        [jnp.concatenate([w, z], axis=1), jnp.concatenate([z, w], axis=1)],
        axis=0)


def kernel(x, edge_index, W1, b1, W2, b2, W3, b3, Wfc, bfc):
    src, dst = edge_index[0], edge_index[1]
    src = 2 * src - jnp.where(src >= _NH, 9999, 0)
    dst = 2 * dst - jnp.where(dst >= _NH, 9999, 0)
    npad = _EPAD - _E
    pidx = jnp.arange(npad, dtype=jnp.int32)
    psrc = jnp.bitwise_and(pidx * 37, 8191)
    pdst = _N + jnp.bitwise_and(pidx, 127)
    src_p = jnp.concatenate([src, psrc]).reshape(_NW, _NCH, _CHW)
    dst_p = jnp.concatenate([dst, pdst]).reshape(_NW, _NCH, _CHW)
    z1d = jnp.zeros((_NPAD,), _f32)
    z2d = jnp.zeros((_RPT, _DH), _f32)
    w2d = _blockdiag(W2)
    w3d = _blockdiag(W3)
    wfcd = _blockdiag(Wfc)
    b1p = jnp.concatenate([b1, b1])[None, :]
    b2p = jnp.concatenate([b2, b2])[None, :]
    b3p = jnp.concatenate([b3, b3])[None, :]
    bfcp = jnp.concatenate([bfc, bfc])[None, :]

    h1 = _mm1(x, W1)
    degs = _sc_deg(dst_p, z1d).reshape(2, _NPAD // 2, 2)
    dinv, g1 = _stage1(degs, h1)

    def s_packed(g):
        s = _sc_scatter(src_p, dst_p, g.reshape(_N, _DH), z2d)
        return s.reshape(2, _NPAD // 2, 2 * _DH)

    g2 = _mid(s_packed(g1), g1, dinv, b1p, w2d)
    g3 = _mid(s_packed(g2), g2, dinv, b2p, w3d)
    outp = _final(s_packed(g3), g3, dinv, b3p, wfcd, bfcp)
    o = outp.reshape(_NH, 2, _DOUT)
    return jnp.concatenate([o[:, 0], o[:, 1]], axis=0)
